# baseline (speedup 1.0000x reference)
import functools

import jax
import jax.numpy as jnp
from jax import lax
from jax.experimental import pallas as pl
from jax.experimental.pallas import tpu as pltpu
from jax.experimental.pallas import tpu_sc as plsc

NC = 2
NS = 16
NW = NC * NS
K = 128
NBUF = 2
SUP = 16


def _acc_rows(n_nodes):
  return ((n_nodes + 1 + NS * 8 - 1) // (NS * 8)) * (NS * 8)


def _sc_segment_sum(n_nodes, d, ch):
  np_rows = _acc_rows(n_nodes)
  zrows = np_rows // NS

  mesh = plsc.VectorSubcoreMesh(
      core_axis_name="c", subcore_axis_name="s", num_cores=NC,
      num_subcores=NS)

  n_sets = ch // SUP

  @functools.partial(
      pl.kernel,
      out_type=jax.ShapeDtypeStruct((NC, np_rows, d), jnp.float32),
      mesh=mesh,
      scratch_types=[
          [pltpu.VMEM((SUP, K), jnp.int32)] * 2,
          [pltpu.VMEM((SUP, K), jnp.int32)] * 2,
          [pltpu.VMEM((K, d), jnp.float32)] * NBUF,
          [pltpu.SemaphoreType.DMA] * NBUF,
          pltpu.SemaphoreType.DMA,
          pltpu.VMEM_SHARED((np_rows, d), jnp.float32),
      ],
  )
  def seg_sum(x_hbm, src_hbm, dst_hbm, z_hbm, out_hbm,
              src_sets, dst_sets, rows_bufs, sems, isem, agg_sh):
    c = lax.axis_index("c")
    s = lax.axis_index("s")
    wid = c * NS + s

    pltpu.sync_copy(z_hbm, agg_sh.at[pl.ds(s * zrows, zrows)])
    plsc.subcore_barrier()

    def refill(si, p, wait):
      rows = pl.ds(si * SUP, SUP)
      for hbm, vmem in ((src_hbm, src_sets[p]), (dst_hbm, dst_sets[p])):
        cp = pltpu.make_async_copy(hbm.at[wid].at[rows], vmem, isem)
        if wait:
          cp.wait()
        else:
          cp.start()

    def gather(sref, jl, b):
      pltpu.async_copy(x_hbm.at[sref.at[jl]], rows_bufs[b], sems[b])

    def gather_wait(sref, jl, b):
      pltpu.make_async_copy(x_hbm.at[sref.at[jl]], rows_bufs[b],
                            sems[b]).wait()

    def consume(dref, jl, b):
      pltpu.sync_copy(rows_bufs[b], agg_sh.at[dref.at[jl]], add=True)

    refill(0, 0, wait=False)
    refill(0, 0, wait=True)
    if n_sets > 1:
      refill(1, 1, wait=False)
    for b in range(NBUF):
      gather(src_sets[0], b, b)

    for si in range(n_sets):
      p = si % 2
      sp, dp = src_sets[p], dst_sets[p]
      if si >= 1 and si + 1 < n_sets:
        refill(si + 1, 1 - p, wait=False)

      def body(jj, carry, sp=sp, dp=dp):
        for b in range(NBUF):
          jl = jj * NBUF + b
          gather_wait(sp, jl, b)
          consume(dp, jl, b)
          gather(sp, jl + NBUF, b)
        return carry

      lax.fori_loop(0, (SUP - NBUF) // NBUF, body, 0)

      if si + 1 < n_sets:
        refill(si + 1, 1 - p, wait=True)
      for b in range(NBUF):
        jl = SUP - NBUF + b
        gather_wait(sp, jl, b)
        consume(dp, jl, b)
        if si + 1 < n_sets:
          gather(src_sets[1 - p], b, b)

    plsc.subcore_barrier()

    r0 = s * zrows
    pltpu.sync_copy(agg_sh.at[pl.ds(r0, zrows)],
                    out_hbm.at[c].at[pl.ds(r0, zrows)])

  return seg_sum


def _tc_finish_body(x_ref, agg_ref, u_ref, v_ref, o_ref):
  agg = agg_ref[0] + agg_ref[1]
  h = (jnp.dot(x_ref[...], u_ref[...], preferred_element_type=jnp.float32)
       + jnp.dot(agg, v_ref[...], preferred_element_type=jnp.float32))
  h = jnp.maximum(h, 0.0)
  m = jnp.max(h, axis=-1, keepdims=True)
  e = jnp.exp(h - m)
  o_ref[...] = e / jnp.sum(e, axis=-1, keepdims=True)


def kernel(x, edge_index, U, V):
  n, d = x.shape
  e = edge_index.shape[1]

  per_tile = (e + NW * K * SUP - 1) // (NW * K * SUP) * (K * SUP)
  ch = per_tile // K
  e_pad = per_tile * NW
  pad = e_pad - e
  src = jnp.concatenate([edge_index[0], jnp.zeros((pad,), jnp.int32)])
  dst = jnp.concatenate([edge_index[1], jnp.full((pad,), n, jnp.int32)])
  src = src.reshape(NW, ch, K)
  dst = dst.reshape(NW, ch, K)

  np_rows = _acc_rows(n)
  zeros = jnp.zeros((np_rows // NS, d), jnp.float32)

  agg2 = _sc_segment_sum(n, d, ch)(x, src, dst, zeros)

  blk = 1000
  grid = n // blk
  out = pl.pallas_call(
      _tc_finish_body,
      grid=(grid,),
      in_specs=[
          pl.BlockSpec((blk, d), lambda i: (i, 0)),
          pl.BlockSpec((NC, blk, d), lambda i: (0, i, 0)),
          pl.BlockSpec((d, d), lambda i: (0, 0)),
          pl.BlockSpec((d, d), lambda i: (0, 0)),
      ],
      out_specs=pl.BlockSpec((blk, d), lambda i: (i, 0)),
      out_shape=jax.ShapeDtypeStruct((n, d), jnp.float32),
  )(x, agg2, U, V)
  return out


# speedup vs baseline: 3.7346x; 3.7346x over previous
import functools

import jax
import jax.numpy as jnp
from jax import lax
from jax.experimental import pallas as pl
from jax.experimental.pallas import tpu as pltpu
from jax.experimental.pallas import tpu_sc as plsc

NC = 2
NS = 16
NW = NC * NS
K = 128
NBUF = 2
SUP = 16


def _acc_rows(n_nodes):
  return ((n_nodes + 1 + NS * 8 - 1) // (NS * 8)) * (NS * 8)


def _sc_segment_sum(n_nodes, d, ch):
  np_rows = _acc_rows(n_nodes)
  zrows = np_rows // NS

  mesh = plsc.VectorSubcoreMesh(
      core_axis_name="c", subcore_axis_name="s", num_cores=NC,
      num_subcores=NS)

  n_sets = ch // SUP

  @functools.partial(
      pl.kernel,
      out_type=jax.ShapeDtypeStruct((NC, np_rows, d), jnp.float32),
      mesh=mesh,
      scratch_types=[
          [pltpu.VMEM((SUP, K), jnp.int32)] * 2,
          [pltpu.VMEM((SUP, K), jnp.int32)] * 2,
          [pltpu.VMEM((K, d), jnp.float32)] * NBUF,
          [pltpu.SemaphoreType.DMA] * NBUF,
          pltpu.SemaphoreType.DMA,
          pltpu.VMEM_SHARED((np_rows, d), jnp.float32),
      ],
  )
  def seg_sum(x_hbm, src_hbm, dst_hbm, z_hbm, out_hbm,
              src_sets, dst_sets, rows_bufs, sems, isem, agg_sh):
    c = lax.axis_index("c")
    s = lax.axis_index("s")
    wid = c * NS + s

    pltpu.sync_copy(z_hbm, agg_sh.at[pl.ds(s * zrows, zrows)])
    plsc.subcore_barrier()

    def refill(si, p, wait):
      rows = pl.ds(si * SUP, SUP)
      for hbm, vmem in ((src_hbm, src_sets[p]), (dst_hbm, dst_sets[p])):
        cp = pltpu.make_async_copy(hbm.at[wid].at[rows], vmem, isem)
        if wait:
          cp.wait()
        else:
          cp.start()

    def gather(sref, jl, b):
      pltpu.async_copy(x_hbm.at[sref.at[jl]], rows_bufs[b], sems[b])

    def gather_wait(sref, jl, b):
      pltpu.make_async_copy(x_hbm.at[sref.at[jl]], rows_bufs[b],
                            sems[b]).wait()

    def consume(dref, jl, b):
      pltpu.sync_copy(rows_bufs[b], agg_sh.at[dref.at[jl]], add=True)

    refill(0, 0, wait=False)
    refill(0, 0, wait=True)
    if n_sets > 1:
      refill(1, 1, wait=False)
    for b in range(NBUF):
      gather(src_sets[0], b, b)

    for si in range(n_sets):
      p = si % 2
      sp, dp = src_sets[p], dst_sets[p]
      if si >= 1 and si + 1 < n_sets:
        refill(si + 1, 1 - p, wait=False)

      def body(jj, carry, sp=sp, dp=dp):
        for b in range(NBUF):
          jl = jj * NBUF + b
          gather_wait(sp, jl, b)
          consume(dp, jl, b)
          gather(sp, jl + NBUF, b)
        return carry

      lax.fori_loop(0, (SUP - NBUF) // NBUF, body, 0)

      if si + 1 < n_sets:
        refill(si + 1, 1 - p, wait=True)
      for b in range(NBUF):
        jl = SUP - NBUF + b
        gather_wait(sp, jl, b)
        consume(dp, jl, b)
        if si + 1 < n_sets:
          gather(src_sets[1 - p], b, b)

    plsc.subcore_barrier()

    r0 = s * zrows
    pltpu.sync_copy(agg_sh.at[pl.ds(r0, zrows)],
                    out_hbm.at[c].at[pl.ds(r0, zrows)])

  return seg_sum


def _tc_finish_body(x_ref, agg_ref, u_ref, v_ref, o_ref):
  agg = agg_ref[0] + agg_ref[1]
  h = (jnp.dot(x_ref[...], u_ref[...], preferred_element_type=jnp.float32)
       + jnp.dot(agg, v_ref[...], preferred_element_type=jnp.float32))
  h = jnp.maximum(h, 0.0)
  m = jnp.max(h, axis=-1, keepdims=True)
  e = jnp.exp(h - m)
  o_ref[...] = e / jnp.sum(e, axis=-1, keepdims=True)


def kernel(x, edge_index, U, V):
  n, d = x.shape
  e = edge_index.shape[1]

  per_tile = (e + NW * K * SUP - 1) // (NW * K * SUP) * (K * SUP)
  ch = per_tile // K
  e_pad = per_tile * NW
  pad = e_pad - e
  np_rows = _acc_rows(n)
  iota = jnp.arange(pad, dtype=jnp.int32)
  src = jnp.concatenate([edge_index[0], iota % n])
  dst = jnp.concatenate([edge_index[1], n + iota % (np_rows - n)])
  src = src.reshape(NW, ch, K)
  dst = dst.reshape(NW, ch, K)

  zeros = jnp.zeros((np_rows // NS, d), jnp.float32)

  agg2 = _sc_segment_sum(n, d, ch)(x, src, dst, zeros)

  blk = 1000
  grid = n // blk
  out = pl.pallas_call(
      _tc_finish_body,
      grid=(grid,),
      in_specs=[
          pl.BlockSpec((blk, d), lambda i: (i, 0)),
          pl.BlockSpec((NC, blk, d), lambda i: (0, i, 0)),
          pl.BlockSpec((d, d), lambda i: (0, 0)),
          pl.BlockSpec((d, d), lambda i: (0, 0)),
      ],
      out_specs=pl.BlockSpec((blk, d), lambda i: (i, 0)),
      out_shape=jax.ShapeDtypeStruct((n, d), jnp.float32),
  )(x, agg2, U, V)
  return out


# speedup vs baseline: 4.1572x; 1.1131x over previous
import functools

import jax
import jax.numpy as jnp
from jax import lax
from jax.experimental import pallas as pl
from jax.experimental.pallas import tpu as pltpu
from jax.experimental.pallas import tpu_sc as plsc

NC = 2
NS = 16
NW = NC * NS
K = 128
NBUF = 2


def _acc_rows(n_nodes):
  return ((n_nodes + 1 + NS * 8 - 1) // (NS * 8)) * (NS * 8)


def _sc_segment_sum(n_nodes, d, e):
  assert e % K == 0
  chunks = e // K
  full = chunks // NW
  extra = chunks % NW
  np_rows = _acc_rows(n_nodes)
  zrows = np_rows // NS

  mesh = plsc.VectorSubcoreMesh(
      core_axis_name="c", subcore_axis_name="s", num_cores=NC,
      num_subcores=NS)

  scratch = [
      [pltpu.VMEM((K,), jnp.int32)] * NBUF,
      [pltpu.VMEM((K,), jnp.int32)] * NBUF,
      [pltpu.VMEM((K, d), jnp.float32)] * NBUF,
      [pltpu.SemaphoreType.DMA] * NBUF,
      [pltpu.SemaphoreType.DMA] * NBUF,
      pltpu.VMEM_SHARED((np_rows, d), jnp.float32),
  ]

  @functools.partial(
      pl.kernel,
      out_type=jax.ShapeDtypeStruct((NC, np_rows, d), jnp.float32),
      mesh=mesh,
      scratch_types=scratch,
  )
  def seg_sum(x_hbm, ei_hbm, z_hbm, out_hbm,
              sidx, didx, rows_bufs, gsem, isem, agg_sh):
    c = lax.axis_index("c")
    s = lax.axis_index("s")
    wid = c * NS + s

    pltpu.sync_copy(z_hbm, agg_sh.at[pl.ds(s * zrows, zrows)])
    plsc.subcore_barrier()

    def fetch(j, b):
      ofs = pl.ds((wid + j * NW) * K, K)
      pltpu.async_copy(ei_hbm.at[0].at[ofs], sidx[b], isem[b])
      pltpu.async_copy(ei_hbm.at[1].at[ofs], didx[b], isem[b])

    def fetch_wait(j, b):
      ofs = pl.ds((wid + j * NW) * K, K)
      pltpu.make_async_copy(ei_hbm.at[0].at[ofs], sidx[b], isem[b]).wait()
      pltpu.make_async_copy(ei_hbm.at[1].at[ofs], didx[b], isem[b]).wait()

    def gather(b):
      pltpu.async_copy(x_hbm.at[sidx[b]], rows_bufs[b], gsem[b])

    def gather_wait(b):
      pltpu.make_async_copy(x_hbm.at[sidx[b]], rows_bufs[b],
                            gsem[b]).wait()

    for b in range(NBUF):
      fetch(b, b)
    for b in range(NBUF):
      fetch_wait(b, b)
      gather(b)

    def body(jj, carry):
      for b in range(NBUF):
        j = jj * NBUF + b
        nxt = j + NBUF

        @pl.when(nxt < full)
        def _(nxt=nxt, b=b):
          fetch(nxt, b)

        gather_wait(b)
        pltpu.sync_copy(rows_bufs[b], agg_sh.at[didx[b]], add=True)

        @pl.when(nxt < full)
        def _(nxt=nxt, b=b):
          fetch_wait(nxt, b)
          gather(b)

      return carry

    assert full % NBUF == 0
    lax.fori_loop(0, full // NBUF, body, 0)

    if extra:
      @pl.when(wid < extra)
      def _():
        fetch(full, 0)
        fetch_wait(full, 0)
        gather(0)
        gather_wait(0)
        pltpu.sync_copy(rows_bufs[0], agg_sh.at[didx[0]], add=True)

    plsc.subcore_barrier()

    r0 = s * zrows
    pltpu.sync_copy(agg_sh.at[pl.ds(r0, zrows)],
                    out_hbm.at[c].at[pl.ds(r0, zrows)])

  return seg_sum


def _tc_finish_body(x_ref, agg_ref, u_ref, v_ref, o_ref):
  agg = agg_ref[0] + agg_ref[1]
  h = (jnp.dot(x_ref[...], u_ref[...], preferred_element_type=jnp.float32)
       + jnp.dot(agg, v_ref[...], preferred_element_type=jnp.float32))
  h = jnp.maximum(h, 0.0)
  m = jnp.max(h, axis=-1, keepdims=True)
  e = jnp.exp(h - m)
  o_ref[...] = e / jnp.sum(e, axis=-1, keepdims=True)


def kernel(x, edge_index, U, V):
  n, d = x.shape
  e = edge_index.shape[1]

  np_rows = _acc_rows(n)
  zeros = jnp.zeros((np_rows // NS, d), jnp.float32)

  agg2 = _sc_segment_sum(n, d, e)(x, edge_index, zeros)

  blk = 1000
  grid = n // blk
  out = pl.pallas_call(
      _tc_finish_body,
      grid=(grid,),
      in_specs=[
          pl.BlockSpec((blk, d), lambda i: (i, 0)),
          pl.BlockSpec((NC, blk, d), lambda i: (0, i, 0)),
          pl.BlockSpec((d, d), lambda i: (0, 0)),
          pl.BlockSpec((d, d), lambda i: (0, 0)),
      ],
      out_specs=pl.BlockSpec((blk, d), lambda i: (i, 0)),
      out_shape=jax.ShapeDtypeStruct((n, d), jnp.float32),
  )(x, agg2, U, V)
  return out
